# Initial kernel scaffold; baseline (speedup 1.0000x reference)
import jax
import jax.numpy as jnp
from jax.experimental import pallas as pl


def kernel(x, w_0_0, b_0_0, w_0_1, b_0_1, w_0_2, b_0_2, w_0_3, b_0_3, w_0_4, b_0_4, w_1_0, b_1_0, w_1_1, b_1_1, w_1_2, b_1_2, w_1_3, b_1_3, w_1_4, b_1_4, w_2_0, b_2_0, w_2_1, b_2_1, w_2_2, b_2_2, w_2_3, b_2_3, w_2_4, b_2_4):
    raise NotImplementedError("write your pallas kernel here")


# speedup vs baseline: 2.1828x; 2.1828x over previous
import functools

import numpy as np
import jax
import jax.numpy as jnp
from jax.experimental import pallas as pl
from jax.experimental.pallas import tpu as pltpu


def _s2_conv_kernel(x_ref, w_ref, b_ref, o_ref, *, Wi, slope, fuse_k):
    p_out = o_ref.shape[0]
    shifts = (0, 1, Wi, Wi + 1)
    ce = x_ref.shape[1]
    if fuse_k:
        xc = jnp.concatenate([x_ref[s:s + p_out, :] for s in shifts], axis=1)
        acc = jnp.dot(xc, w_ref[...], preferred_element_type=jnp.float32)
    else:
        acc = None
        for t, s in enumerate(shifts):
            part = jnp.dot(x_ref[s:s + p_out, :], w_ref[t * ce:(t + 1) * ce, :],
                           preferred_element_type=jnp.float32)
            acc = part if acc is None else acc + part
    y = acc + b_ref[0]
    y = jnp.where(y >= 0.0, y, slope * y)
    o_ref[...] = y.astype(o_ref.dtype)


def _conv34_kernel(x_ref, w3_ref, b3_ref, w4_ref, b4_ref, o_ref, x4_ref, *,
                   Wi3, Ho3, Wo3, slope):
    p3_out = Ho3 * Wi3
    shifts3 = tuple(dy * Wi3 + dx for dy in range(4) for dx in range(4))
    xc = jnp.concatenate([x_ref[s:s + p3_out, :] for s in shifts3], axis=1)
    y3 = jnp.dot(xc, w3_ref[...], preferred_element_type=jnp.float32)
    y3 = y3 + b3_ref[0]
    y3 = jnp.where(y3 >= 0.0, y3, slope * y3).astype(jnp.bfloat16)

    h4 = Ho3
    wi4 = Wo3 + 4
    x4_ref[...] = jnp.zeros(x4_ref.shape, x4_ref.dtype)
    for r in range(Ho3):
        x4_ref[(r + 2) * wi4 + 2:(r + 2) * wi4 + 2 + Wo3, :] = \
            y3[r * Wi3:r * Wi3 + Wo3, :]

    p4_out = (h4 + 1) * wi4
    x4 = x4_ref[...]
    t2 = jax.lax.dot_general(w4_ref[...], x4, (((1,), (1,)), ((), ())),
                             preferred_element_type=jnp.float32)
    acc = None
    for t, s in enumerate(dy * wi4 + dx for dy in range(4) for dx in range(4)):
        part = t2[t:t + 1, s:s + p4_out]
        acc = part if acc is None else acc + part
    o_ref[...] = acc + b4_ref[0, 0]


def _pool_mm_kernel(m_ref, x_ref, o_ref):
    o_ref[...] = jnp.dot(m_ref[...], x_ref[...],
                         preferred_element_type=jnp.float32)


def _s2_conv(x, w, b, *, slope):
    N, H, W, Cin = x.shape
    Cout = w.shape[-1]
    Ho, Wo = H // 2 + 1, W // 2 + 1
    eh, ew = (H + 4) % 2, (W + 4) % 2
    xp = jnp.pad(x, ((0, 0), (2, 2 + eh), (2, 2 + ew), (0, 0)))
    Hp, Wp = xp.shape[1], xp.shape[2]
    xin = xp.reshape(N, Hp // 2, 2, Wp // 2, 2, Cin)
    xin = xin.transpose(0, 1, 3, 2, 4, 5).reshape(N, Hp // 2, Wp // 2, 4 * Cin)
    w_taps = (w.reshape(2, 2, 2, 2, Cin, Cout)
               .transpose(0, 2, 1, 3, 4, 5)
               .reshape(4 * 4 * Cin, Cout))

    Hi, Wi, Ce = xin.shape[1], xin.shape[2], xin.shape[3]
    Hk = Hi - 1
    xin = jnp.pad(xin, ((0, 0), (0, 1), (0, 0), (0, 0)))
    x_flat = xin.reshape(N, (Hi + 1) * Wi, Ce).astype(jnp.bfloat16)
    p_in, p_out = (Hi + 1) * Wi, Hk * Wi
    fuse_k = Cin % 32 == 0

    out = pl.pallas_call(
        functools.partial(_s2_conv_kernel, Wi=Wi, slope=slope, fuse_k=fuse_k),
        out_shape=jax.ShapeDtypeStruct((N, p_out, Cout), jnp.bfloat16),
        grid=(N,),
        in_specs=[
            pl.BlockSpec((None, p_in, Ce), lambda n: (n, 0, 0)),
            pl.BlockSpec((4 * Ce, Cout), lambda n: (0, 0)),
            pl.BlockSpec((1, Cout), lambda n: (0, 0)),
        ],
        out_specs=pl.BlockSpec((None, p_out, Cout), lambda n: (n, 0, 0)),
        compiler_params=pltpu.CompilerParams(
            dimension_semantics=("parallel",)),
    )(x_flat, w_taps.astype(jnp.bfloat16), b.reshape(1, Cout))
    return out.reshape(N, Hk, Wi, Cout)[:, :Ho, :Wo, :]


def _conv34(x, w3, b3, w4, b4, *, slope):
    N, H3, W3, C3 = x.shape
    C4 = w3.shape[-1]
    Hi3, Wi3 = H3 + 4, W3 + 4
    Ho3, Wo3 = H3 + 1, W3 + 1
    xp = jnp.pad(x, ((0, 0), (2, 2), (2, 2), (0, 0)))
    xp = jnp.pad(xp, ((0, 0), (0, 1), (0, 0), (0, 0)))
    x_flat = xp.reshape(N, (Hi3 + 1) * Wi3, C3).astype(jnp.bfloat16)
    p3_in = (Hi3 + 1) * Wi3

    h4, wi4 = Ho3, Wo3 + 4
    p4_in = (h4 + 5) * wi4
    p4_out = (h4 + 1) * wi4
    Ho4, Wo4 = h4 + 1, Wo3 + 1

    w3_cat = w3.reshape(16 * C3, C4).astype(jnp.bfloat16)
    w4_rows = w4.reshape(16, C4).astype(jnp.bfloat16)

    out = pl.pallas_call(
        functools.partial(_conv34_kernel, Wi3=Wi3, Ho3=Ho3, Wo3=Wo3,
                          slope=slope),
        out_shape=jax.ShapeDtypeStruct((N, 1, p4_out), jnp.float32),
        grid=(N,),
        in_specs=[
            pl.BlockSpec((None, p3_in, C3), lambda n: (n, 0, 0)),
            pl.BlockSpec((16 * C3, C4), lambda n: (0, 0)),
            pl.BlockSpec((1, C4), lambda n: (0, 0)),
            pl.BlockSpec((16, C4), lambda n: (0, 0)),
            pl.BlockSpec((1, 1), lambda n: (0, 0)),
        ],
        out_specs=pl.BlockSpec((None, 1, p4_out), lambda n: (n, 0, 0)),
        scratch_shapes=[pltpu.VMEM((p4_in, C4), jnp.bfloat16)],
        compiler_params=pltpu.CompilerParams(
            dimension_semantics=("parallel",)),
    )(x_flat, w3_cat, b3.reshape(1, C4), w4_rows, b4.reshape(1, 1))
    return out.reshape(N, h4 + 1, wi4)[:, :Ho4, :Wo4, None]


def _pool1d(n):
    no = (n - 1) // 2 + 1
    p = np.zeros((no, n), np.float32)
    for o in range(no):
        cols = [c for c in (2 * o - 1, 2 * o, 2 * o + 1) if 0 <= c < n]
        p[o, cols] = 1.0 / len(cols)
    return p


def _avgpool(x):
    N, H, W, C = x.shape
    ph, pw = _pool1d(H), _pool1d(W)
    Ho, Wo = ph.shape[0], pw.shape[0]
    m = jnp.asarray(np.kron(ph, pw), dtype=jnp.bfloat16)
    xt = x.transpose(1, 2, 0, 3).reshape(H * W, N * C).astype(jnp.bfloat16)
    mo = Ho * Wo
    out = pl.pallas_call(
        _pool_mm_kernel,
        out_shape=jax.ShapeDtypeStruct((mo, N * C), jnp.float32),
        grid=(2,),
        in_specs=[
            pl.BlockSpec((mo // 2, H * W), lambda i: (i, 0)),
            pl.BlockSpec((H * W, N * C), lambda i: (0, 0)),
        ],
        out_specs=pl.BlockSpec((mo // 2, N * C), lambda i: (i, 0)),
        compiler_params=pltpu.CompilerParams(
            dimension_semantics=("parallel",)),
    )(m, xt)
    return out.reshape(Ho, Wo, N, C).transpose(2, 0, 1, 3)


def _discriminator(x, ws, bs):
    h = _s2_conv(x, ws[0], bs[0], slope=0.2)
    h = _s2_conv(h, ws[1], bs[1], slope=0.2)
    h = _s2_conv(h, ws[2], bs[2], slope=0.2)
    return _conv34(h, ws[3], bs[3], ws[4], bs[4], slope=0.2)


def kernel(x, w_0_0, b_0_0, w_0_1, b_0_1, w_0_2, b_0_2, w_0_3, b_0_3, w_0_4, b_0_4,
           w_1_0, b_1_0, w_1_1, b_1_1, w_1_2, b_1_2, w_1_3, b_1_3, w_1_4, b_1_4,
           w_2_0, b_2_0, w_2_1, b_2_1, w_2_2, b_2_2, w_2_3, b_2_3, w_2_4, b_2_4):
    Ws = [[w_0_0, w_0_1, w_0_2, w_0_3, w_0_4],
          [w_1_0, w_1_1, w_1_2, w_1_3, w_1_4],
          [w_2_0, w_2_1, w_2_2, w_2_3, w_2_4]]
    Bs = [[b_0_0, b_0_1, b_0_2, b_0_3, b_0_4],
          [b_1_0, b_1_1, b_1_2, b_1_3, b_1_4],
          [b_2_0, b_2_1, b_2_2, b_2_3, b_2_4]]
    results = []
    inp = x
    for i in range(3):
        results.append(_discriminator(inp, Ws[2 - i], Bs[2 - i]))
        if i != 2:
            inp = _avgpool(inp)
    return results


# speedup vs baseline: 2.9954x; 1.3723x over previous
import functools
from types import SimpleNamespace

import numpy as np
import jax
import jax.numpy as jnp
from jax.experimental import pallas as pl
from jax.experimental.pallas import tpu as pltpu


def _lrelu(y, slope):
    return jnp.where(y >= 0.0, y, slope * y)


def _quadrant_flats(P, Hq, Wh, C):
    out = []
    for a in range(2):
        Pa = P.reshape(Hq, 2, 2 * Wh, C)[:, a]
        for b in range(2):
            Q = Pa.reshape(Hq, Wh, 2, C)[:, :, b, :]
            out.append(Q.reshape(Hq * Wh, C))
    return out


def _s2_conv_block(P, Hq, Wh, C, p_out, w_ref, b_ref, slope):
    qs = _quadrant_flats(P, Hq, Wh, C)
    xcat = jnp.concatenate(
        [qs[g][s:s + p_out, :] for s in (0, 1, Wh, Wh + 1) for g in range(4)],
        axis=1)
    y = jnp.dot(xcat, w_ref[...], preferred_element_type=jnp.float32)
    y = _lrelu(y + b_ref[0], slope)
    return y.astype(jnp.bfloat16)


def _scatter_pad3(dst3, y, Wi, Ho, Wo):
    dst3[...] = jnp.zeros(dst3.shape, dst3.dtype)
    for r in range(Ho):
        dst3[r + 2, 2:2 + Wo, :] = y[r * Wi:r * Wi + Wo, :]


def _scale_body(x_ref, w0, b0, w1, b1, w2, b2, w3, b3, w4, b4, o_ref,
                P1, P2, X3, X4, *, D, slope=0.2):
    p0_out = D.Hk0 * D.Wh0
    acc = None
    for t, s in enumerate((0, 1, D.Wh0, D.Wh0 + 1)):
        part = jnp.dot(x_ref[s:s + p0_out, :], w0[t * 12:(t + 1) * 12, :],
                       preferred_element_type=jnp.float32)
        acc = part if acc is None else acc + part
    v0 = _lrelu(acc + b0[0], slope).astype(jnp.bfloat16)

    _scatter_pad3(P1, v0, D.Wh0, D.Ho0, D.Ho0)
    v1 = _s2_conv_block(P1[...], D.Hq1, D.Wh1, 64, D.p1_out, w1, b1, slope)

    _scatter_pad3(P2, v1, D.Wh1, D.Ho1, D.Ho1)
    v2 = _s2_conv_block(P2[...], D.Hq2, D.Wh2, 128, D.p2_out, w2, b2, slope)

    X3[...] = jnp.zeros(X3.shape, X3.dtype)
    for r in range(D.Ho2):
        X3[(r + 2) * D.Wi3 + 2:(r + 2) * D.Wi3 + 2 + D.Ho2, :] = \
            v2[r * D.Wh2:r * D.Wh2 + D.Ho2, :]
    x3 = X3[...]
    p3_out = D.Ho3 * D.Wi3
    shifts3 = tuple(dy * D.Wi3 + dx for dy in range(4) for dx in range(4))
    xc3 = jnp.concatenate([x3[s:s + p3_out, :] for s in shifts3], axis=1)
    y3 = jnp.dot(xc3, w3[...], preferred_element_type=jnp.float32)
    y3 = _lrelu(y3 + b3[0], slope).astype(jnp.bfloat16)

    wi4 = D.Wo3 + 4
    X4[...] = jnp.zeros(X4.shape, X4.dtype)
    for r in range(D.Ho3):
        X4[(r + 2) * wi4 + 2:(r + 2) * wi4 + 2 + D.Wo3, :] = \
            y3[r * D.Wi3:r * D.Wi3 + D.Wo3, :]
    p4_out = (D.Ho3 + 1) * wi4
    t2 = jax.lax.dot_general(w4[...], X4[...], (((1,), (1,)), ((), ())),
                             preferred_element_type=jnp.float32)
    acc4 = None
    for t, s in enumerate(dy * wi4 + dx for dy in range(4) for dx in range(4)):
        part = t2[t:t + 1, s:s + p4_out]
        acc4 = part if acc4 is None else acc4 + part
    o_ref[...] = acc4 + b4[0, 0]


def _dims(S):
    D = SimpleNamespace()
    D.S = S
    D.Hh0 = (S + 4) // 2
    D.Wh0 = D.Hh0
    D.Hk0 = D.Hh0 - 1
    D.Ho0 = S // 2 + 1
    D.Hq1 = (D.Ho0 + 5) // 2 + 1
    D.Wh1 = (D.Ho0 + 5) // 2
    D.p1_out = (D.Wh1 - 1) * D.Wh1
    D.Ho1 = D.Ho0 // 2 + 1
    D.Hq2 = (D.Ho1 + 5) // 2 + 1
    D.Wh2 = (D.Ho1 + 5) // 2
    D.p2_out = (D.Wh2 - 1) * D.Wh2
    D.Ho2 = D.Ho1 // 2 + 1
    D.Wi3 = D.Ho2 + 4
    D.p3_in = (D.Ho2 + 5) * D.Wi3
    D.Ho3 = D.Ho2 + 1
    D.Wo3 = D.Ho2 + 1
    D.wi4 = D.Wo3 + 4
    D.p4_in = (D.Ho3 + 5) * D.wi4
    D.p4_out = (D.Ho3 + 1) * D.wi4
    D.Ho4 = D.Ho3 + 1
    D.Wo4 = D.Wo3 + 1
    return D


def _prep_conv0(x):
    N, H, W, Cin = x.shape
    xp = jnp.pad(x, ((0, 0), (2, 2), (2, 2), (0, 0)))
    Hp = xp.shape[1]
    xin = xp.reshape(N, Hp // 2, 2, Hp // 2, 2, Cin)
    xin = xin.transpose(0, 1, 3, 2, 4, 5).reshape(N, Hp // 2, Hp // 2, 4 * Cin)
    xin = jnp.pad(xin, ((0, 0), (0, 1), (0, 0), (0, 0)))
    Hh = Hp // 2
    return xin.reshape(N, (Hh + 1) * Hh, 4 * Cin).astype(jnp.bfloat16)


def _w_s2(w):
    cin, cout = w.shape[2], w.shape[3]
    return (w.reshape(2, 2, 2, 2, cin, cout)
             .transpose(0, 2, 1, 3, 4, 5)
             .reshape(16 * cin, cout).astype(jnp.bfloat16))


def _run_scale(x, ws, bs):
    N = x.shape[0]
    D = _dims(x.shape[1])
    x_flat = _prep_conv0(x)
    p0_in = x_flat.shape[1]

    w0 = _w_s2(ws[0])
    w1 = _w_s2(ws[1])
    w2 = _w_s2(ws[2])
    w3 = ws[3].reshape(16 * 256, 512).astype(jnp.bfloat16)
    w4 = ws[4].reshape(16, 512).astype(jnp.bfloat16)
    b = [bs[j].reshape(1, -1) for j in range(5)]

    out = pl.pallas_call(
        functools.partial(_scale_body, D=D),
        out_shape=jax.ShapeDtypeStruct((N, 1, D.p4_out), jnp.float32),
        grid=(N,),
        in_specs=[
            pl.BlockSpec((None, p0_in, 12), lambda n: (n, 0, 0)),
            pl.BlockSpec(w0.shape, lambda n: (0, 0)),
            pl.BlockSpec(b[0].shape, lambda n: (0, 0)),
            pl.BlockSpec(w1.shape, lambda n: (0, 0)),
            pl.BlockSpec(b[1].shape, lambda n: (0, 0)),
            pl.BlockSpec(w2.shape, lambda n: (0, 0)),
            pl.BlockSpec(b[2].shape, lambda n: (0, 0)),
            pl.BlockSpec(w3.shape, lambda n: (0, 0)),
            pl.BlockSpec(b[3].shape, lambda n: (0, 0)),
            pl.BlockSpec(w4.shape, lambda n: (0, 0)),
            pl.BlockSpec(b[4].shape, lambda n: (0, 0)),
        ],
        out_specs=pl.BlockSpec((None, 1, D.p4_out), lambda n: (n, 0, 0)),
        scratch_shapes=[
            pltpu.VMEM((2 * D.Hq1, 2 * D.Wh1, 64), jnp.bfloat16),
            pltpu.VMEM((2 * D.Hq2, 2 * D.Wh2, 128), jnp.bfloat16),
            pltpu.VMEM((D.p3_in, 256), jnp.bfloat16),
            pltpu.VMEM((D.p4_in, 512), jnp.bfloat16),
        ],
        compiler_params=pltpu.CompilerParams(
            dimension_semantics=("parallel",)),
    )(x_flat, w0, b[0], w1, b[1], w2, b[2], w3, b[3], w4, b[4])
    return out.reshape(N, D.Ho3 + 1, D.wi4)[:, :D.Ho4, :D.Wo4, None]


def _pool1d(n):
    no = (n - 1) // 2 + 1
    p = np.zeros((no, n), np.float32)
    for o in range(no):
        cols = [c for c in (2 * o - 1, 2 * o, 2 * o + 1) if 0 <= c < n]
        p[o, cols] = 1.0 / len(cols)
    return p


def _pools_kernel(m1_ref, m2_ref, x_ref, o1_ref, o2_ref):
    p1 = jnp.dot(m1_ref[...], x_ref[...], preferred_element_type=jnp.float32)
    o1_ref[...] = p1
    o2_ref[...] = jnp.dot(m2_ref[...], p1.astype(jnp.bfloat16),
                          preferred_element_type=jnp.float32)


def _pools(x):
    N, H, W, C = x.shape
    m1 = jnp.asarray(np.kron(_pool1d(H), _pool1d(W)), dtype=jnp.bfloat16)
    H2 = (H - 1) // 2 + 1
    m2 = jnp.asarray(np.kron(_pool1d(H2), _pool1d(H2)), dtype=jnp.bfloat16)
    xt = x.transpose(1, 2, 0, 3).reshape(H * W, N * C).astype(jnp.bfloat16)
    lanes = N * C
    H3 = (H2 - 1) // 2 + 1
    o1, o2 = pl.pallas_call(
        _pools_kernel,
        out_shape=(jax.ShapeDtypeStruct((H2 * H2, lanes), jnp.float32),
                   jax.ShapeDtypeStruct((H3 * H3, lanes), jnp.float32)),
        grid=(1,),
        in_specs=[
            pl.BlockSpec(m1.shape, lambda i: (0, 0)),
            pl.BlockSpec(m2.shape, lambda i: (0, 0)),
            pl.BlockSpec((H * W, lanes), lambda i: (0, 0)),
        ],
        out_specs=(pl.BlockSpec((H2 * H2, lanes), lambda i: (0, 0)),
                   pl.BlockSpec((H3 * H3, lanes), lambda i: (0, 0))),
        compiler_params=pltpu.CompilerParams(
            dimension_semantics=("arbitrary",)),
    )(m1, m2, xt)
    x2 = o1.reshape(H2, H2, N, C).transpose(2, 0, 1, 3)
    x3 = o2.reshape(H3, H3, N, C).transpose(2, 0, 1, 3)
    return x2, x3


def kernel(x, w_0_0, b_0_0, w_0_1, b_0_1, w_0_2, b_0_2, w_0_3, b_0_3, w_0_4, b_0_4,
           w_1_0, b_1_0, w_1_1, b_1_1, w_1_2, b_1_2, w_1_3, b_1_3, w_1_4, b_1_4,
           w_2_0, b_2_0, w_2_1, b_2_1, w_2_2, b_2_2, w_2_3, b_2_3, w_2_4, b_2_4):
    Ws = [[w_0_0, w_0_1, w_0_2, w_0_3, w_0_4],
          [w_1_0, w_1_1, w_1_2, w_1_3, w_1_4],
          [w_2_0, w_2_1, w_2_2, w_2_3, w_2_4]]
    Bs = [[b_0_0, b_0_1, b_0_2, b_0_3, b_0_4],
          [b_1_0, b_1_1, b_1_2, b_1_3, b_1_4],
          [b_2_0, b_2_1, b_2_2, b_2_3, b_2_4]]
    x2, x3 = _pools(x)
    return [_run_scale(x, Ws[2], Bs[2]),
            _run_scale(x2, Ws[1], Bs[1]),
            _run_scale(x3, Ws[0], Bs[0])]


# speedup vs baseline: 3.1014x; 1.0354x over previous
import functools
from types import SimpleNamespace

import numpy as np
import jax
import jax.numpy as jnp
from jax.experimental import pallas as pl
from jax.experimental.pallas import tpu as pltpu


def _lrelu(y, slope):
    return jnp.where(y >= 0.0, y, slope * y)


def _quadrant_flats(P, Hq, Wh, C):
    out = []
    for a in range(2):
        Pa = P.reshape(Hq, 2, 2 * Wh, C)[:, a]
        for b in range(2):
            Q = Pa.reshape(Hq, Wh, 2, C)[:, :, b, :]
            out.append(Q.reshape(Hq * Wh, C))
    return out


def _s2_conv_block(P, Hq, Wh, C, p_out, w_ref, b_ref, slope):
    qs = _quadrant_flats(P, Hq, Wh, C)
    xcat = jnp.concatenate(
        [qs[g][s:s + p_out, :] for s in (0, 1, Wh, Wh + 1) for g in range(4)],
        axis=1)
    y = jnp.dot(xcat, w_ref[...], preferred_element_type=jnp.float32)
    y = _lrelu(y + b_ref[0], slope)
    return y.astype(jnp.bfloat16)


def _scatter_pad3(dst3, y, Wi, Ho, Wo):
    dst3[...] = jnp.zeros(dst3.shape, dst3.dtype)
    for r in range(Ho):
        dst3[r + 2, 2:2 + Wo, :] = y[r * Wi:r * Wi + Wo, :]


def _scale_body(x_ref, w0, b0, w1, b1, w2, b2, w3, b3, w4, b4, o_ref,
                P1, P2, X3, X4, *, D, slope=0.2):
    p0_out = D.Hk0 * D.Wh0
    acc = None
    for t, s in enumerate((0, 1, D.Wh0, D.Wh0 + 1)):
        part = jnp.dot(x_ref[s:s + p0_out, :], w0[t * 12:(t + 1) * 12, :],
                       preferred_element_type=jnp.float32)
        acc = part if acc is None else acc + part
    v0 = _lrelu(acc + b0[0], slope).astype(jnp.bfloat16)

    _scatter_pad3(P1, v0, D.Wh0, D.Ho0, D.Ho0)
    v1 = _s2_conv_block(P1[...], D.Hq1, D.Wh1, 64, D.p1_out, w1, b1, slope)

    _scatter_pad3(P2, v1, D.Wh1, D.Ho1, D.Ho1)
    v2 = _s2_conv_block(P2[...], D.Hq2, D.Wh2, 128, D.p2_out, w2, b2, slope)

    X3[...] = jnp.zeros(X3.shape, X3.dtype)
    for r in range(D.Ho2):
        X3[(r + 2) * D.Wi3 + 2:(r + 2) * D.Wi3 + 2 + D.Ho2, :] = \
            v2[r * D.Wh2:r * D.Wh2 + D.Ho2, :]
    x3 = X3[...]
    p3_out = D.Ho3 * D.Wi3
    shifts3 = tuple(dy * D.Wi3 + dx for dy in range(4) for dx in range(4))
    xc3 = jnp.concatenate([x3[s:s + p3_out, :] for s in shifts3], axis=1)
    y3 = jnp.dot(xc3, w3[...], preferred_element_type=jnp.float32)
    y3 = _lrelu(y3 + b3[0], slope).astype(jnp.bfloat16)

    wi4 = D.Wo3 + 4
    X4[...] = jnp.zeros(X4.shape, X4.dtype)
    for r in range(D.Ho3):
        X4[(r + 2) * wi4 + 2:(r + 2) * wi4 + 2 + D.Wo3, :] = \
            y3[r * D.Wi3:r * D.Wi3 + D.Wo3, :]
    p4_out = (D.Ho3 + 1) * wi4
    t2 = jax.lax.dot_general(w4[...], X4[...], (((1,), (1,)), ((), ())),
                             preferred_element_type=jnp.float32)
    acc4 = None
    for t, s in enumerate(dy * wi4 + dx for dy in range(4) for dx in range(4)):
        part = t2[t:t + 1, s:s + p4_out]
        acc4 = part if acc4 is None else acc4 + part
    y4 = acc4 + b4[0, 0]
    o_ref[...] = jnp.concatenate(
        [y4[:, r * wi4:r * wi4 + D.Wo4] for r in range(D.Ho4)], axis=0)


def _dims(S):
    D = SimpleNamespace()
    D.S = S
    D.Hh0 = (S + 4) // 2
    D.Wh0 = D.Hh0
    D.Hk0 = D.Hh0 - 1
    D.Ho0 = S // 2 + 1
    D.Hq1 = (D.Ho0 + 5) // 2 + 1
    D.Wh1 = (D.Ho0 + 5) // 2
    D.p1_out = (D.Wh1 - 1) * D.Wh1
    D.Ho1 = D.Ho0 // 2 + 1
    D.Hq2 = (D.Ho1 + 5) // 2 + 1
    D.Wh2 = (D.Ho1 + 5) // 2
    D.p2_out = (D.Wh2 - 1) * D.Wh2
    D.Ho2 = D.Ho1 // 2 + 1
    D.Wi3 = D.Ho2 + 4
    D.p3_in = (D.Ho2 + 5) * D.Wi3
    D.Ho3 = D.Ho2 + 1
    D.Wo3 = D.Ho2 + 1
    D.wi4 = D.Wo3 + 4
    D.p4_in = (D.Ho3 + 5) * D.wi4
    D.p4_out = (D.Ho3 + 1) * D.wi4
    D.Ho4 = D.Ho3 + 1
    D.Wo4 = D.Wo3 + 1
    return D


def _prep_conv0(x):
    N, H, W, Cin = x.shape
    xp = jnp.pad(x, ((0, 0), (2, 2), (2, 2), (0, 0)))
    Hp = xp.shape[1]
    xin = xp.reshape(N, Hp // 2, 2, Hp // 2, 2, Cin)
    xin = xin.transpose(0, 1, 3, 2, 4, 5).reshape(N, Hp // 2, Hp // 2, 4 * Cin)
    xin = jnp.pad(xin, ((0, 0), (0, 1), (0, 0), (0, 0)))
    Hh = Hp // 2
    return xin.reshape(N, (Hh + 1) * Hh, 4 * Cin).astype(jnp.bfloat16)


def _w_s2(w):
    cin, cout = w.shape[2], w.shape[3]
    return (w.reshape(2, 2, 2, 2, cin, cout)
             .transpose(0, 2, 1, 3, 4, 5)
             .reshape(16 * cin, cout).astype(jnp.bfloat16))


def _tri_body(*refs, DS):
    xs = refs[0:3]
    outs = refs[33:36]
    scr = refs[36:]
    for k in range(3):
        wb = refs[3 + 10 * k:13 + 10 * k]
        _scale_body(xs[k], *wb, outs[k], *scr[4 * k:4 * k + 4], D=DS[k])


def _run_scales(xs, Ws, Bs):
    N = xs[0].shape[0]
    DS = [_dims(x.shape[1]) for x in xs]
    x_flats = [_prep_conv0(x) for x in xs]

    operands, in_specs = [], []
    for xf in x_flats:
        p0_in = xf.shape[1]
        operands.append(xf)
        in_specs.append(pl.BlockSpec((None, p0_in, 12), lambda n: (n, 0, 0)))
    for k in range(3):
        ws, bs = Ws[k], Bs[k]
        packed = [_w_s2(ws[0]), bs[0].reshape(1, -1),
                  _w_s2(ws[1]), bs[1].reshape(1, -1),
                  _w_s2(ws[2]), bs[2].reshape(1, -1),
                  ws[3].reshape(16 * 256, 512).astype(jnp.bfloat16),
                  bs[3].reshape(1, -1),
                  ws[4].reshape(16, 512).astype(jnp.bfloat16),
                  bs[4].reshape(1, 1)]
        for a in packed:
            operands.append(a)
            in_specs.append(pl.BlockSpec(a.shape, lambda n: (0,) * a.ndim))

    out_shapes = tuple(jax.ShapeDtypeStruct((N, D.Ho4, D.Wo4), jnp.float32)
                       for D in DS)
    out_specs = tuple(pl.BlockSpec((None, D.Ho4, D.Wo4), lambda n: (n, 0, 0))
                      for D in DS)
    scratch = []
    for D in DS:
        scratch += [pltpu.VMEM((2 * D.Hq1, 2 * D.Wh1, 64), jnp.bfloat16),
                    pltpu.VMEM((2 * D.Hq2, 2 * D.Wh2, 128), jnp.bfloat16),
                    pltpu.VMEM((D.p3_in, 256), jnp.bfloat16),
                    pltpu.VMEM((D.p4_in, 512), jnp.bfloat16)]

    outs = pl.pallas_call(
        functools.partial(_tri_body, DS=DS),
        out_shape=out_shapes,
        grid=(N,),
        in_specs=in_specs,
        out_specs=out_specs,
        scratch_shapes=scratch,
        compiler_params=pltpu.CompilerParams(
            dimension_semantics=("parallel",)),
    )(*operands)
    return [o[..., None] for o in outs]


def _pool1d(n):
    no = (n - 1) // 2 + 1
    p = np.zeros((no, n), np.float32)
    for o in range(no):
        cols = [c for c in (2 * o - 1, 2 * o, 2 * o + 1) if 0 <= c < n]
        p[o, cols] = 1.0 / len(cols)
    return p


def _pools_kernel(m1_ref, m2_ref, x_ref, o1_ref, o2_ref):
    p1 = jnp.dot(m1_ref[...], x_ref[...], preferred_element_type=jnp.float32)
    o1_ref[...] = p1
    o2_ref[...] = jnp.dot(m2_ref[...], p1.astype(jnp.bfloat16),
                          preferred_element_type=jnp.float32)


def _pools(x):
    N, H, W, C = x.shape
    m1 = jnp.asarray(np.kron(_pool1d(H), _pool1d(W)), dtype=jnp.bfloat16)
    H2 = (H - 1) // 2 + 1
    m2 = jnp.asarray(np.kron(_pool1d(H2), _pool1d(H2)), dtype=jnp.bfloat16)
    xt = x.transpose(1, 2, 0, 3).reshape(H * W, N * C).astype(jnp.bfloat16)
    lanes = N * C
    H3 = (H2 - 1) // 2 + 1
    o1, o2 = pl.pallas_call(
        _pools_kernel,
        out_shape=(jax.ShapeDtypeStruct((H2 * H2, lanes), jnp.float32),
                   jax.ShapeDtypeStruct((H3 * H3, lanes), jnp.float32)),
        grid=(1,),
        in_specs=[
            pl.BlockSpec(m1.shape, lambda i: (0, 0)),
            pl.BlockSpec(m2.shape, lambda i: (0, 0)),
            pl.BlockSpec((H * W, lanes), lambda i: (0, 0)),
        ],
        out_specs=(pl.BlockSpec((H2 * H2, lanes), lambda i: (0, 0)),
                   pl.BlockSpec((H3 * H3, lanes), lambda i: (0, 0))),
        compiler_params=pltpu.CompilerParams(
            dimension_semantics=("arbitrary",)),
    )(m1, m2, xt)
    x2 = o1.reshape(H2, H2, N, C).transpose(2, 0, 1, 3)
    x3 = o2.reshape(H3, H3, N, C).transpose(2, 0, 1, 3)
    return x2, x3


def kernel(x, w_0_0, b_0_0, w_0_1, b_0_1, w_0_2, b_0_2, w_0_3, b_0_3, w_0_4, b_0_4,
           w_1_0, b_1_0, w_1_1, b_1_1, w_1_2, b_1_2, w_1_3, b_1_3, w_1_4, b_1_4,
           w_2_0, b_2_0, w_2_1, b_2_1, w_2_2, b_2_2, w_2_3, b_2_3, w_2_4, b_2_4):
    Ws = [[w_0_0, w_0_1, w_0_2, w_0_3, w_0_4],
          [w_1_0, w_1_1, w_1_2, w_1_3, w_1_4],
          [w_2_0, w_2_1, w_2_2, w_2_3, w_2_4]]
    Bs = [[b_0_0, b_0_1, b_0_2, b_0_3, b_0_4],
          [b_1_0, b_1_1, b_1_2, b_1_3, b_1_4],
          [b_2_0, b_2_1, b_2_2, b_2_3, b_2_4]]
    x2, x3 = _pools(x)
    return _run_scales([x, x2, x3],
                       [Ws[2], Ws[1], Ws[0]],
                       [Bs[2], Bs[1], Bs[0]])
